# baseline (speedup 1.0000x reference)
import functools

import jax
import jax.numpy as jnp
from jax import lax
from jax.experimental import pallas as pl
from jax.experimental.pallas import tpu as pltpu
from jax.experimental.pallas import tpu_sc as plsc

NN = 10000
EE = 320000
GG = 64
FF = 128
HD = 32
OUTD = 10

NBR = 3
BPAD = 10240
TPAD = NBR * BPAD
WROWS = 2 * BPAD

NWORK = 32
SUB = 128
KSUB = 4
CHUNK = SUB * KSUB
EPAD = 983040
PERW = EPAD // NWORK
NCH = PERW // CHUNK
EROWS = EPAD // SUB
CORE0E = 16 * PERW
ZROWS = WROWS // 16
ZP = ZROWS // SUB

_f32 = jnp.float32
_i32 = jnp.int32


def _spmm_kernel_body(recon, src_hbm, dst_hbm, val_hbm, tbl_hbm, bias_hbm,
                      out_hbm, srcb0, dstb0, valb0, srcb1, dstb1, valb1,
                      srcb2, dstb2, valb2, rowsb0, rowsb1, tbl_sh, acc,
                      semld0, semld1, semld2, semg0, semg1, sems0, sems1):
    c = lax.axis_index("c")
    s = lax.axis_index("s")
    wid = c * 16 + s
    zbase = s * ZROWS

    zero16 = jnp.zeros((16,), _f32)
    for k in range(KSUB):
        def zbody(j, carry, k=k):
            rowsb0[k, j, pl.ds(0, 16)] = zero16
            rowsb0[k, j, pl.ds(16, 16)] = zero16
            return carry

        lax.fori_loop(0, SUB, zbody, 0)
    for p in range(ZP):
        pltpu.sync_copy(rowsb0.at[p % KSUB],
                        acc.at[pl.ds(zbase + p * SUB, SUB)])

    if not recon:
        wbase = c * BPAD
        for p in range(ZP):
            pltpu.sync_copy(tbl_hbm.at[pl.ds(wbase + zbase + p * SUB, SUB)],
                            rowsb1.at[p % KSUB])
            pltpu.sync_copy(rowsb1.at[p % KSUB],
                            tbl_sh.at[pl.ds(zbase + p * SUB, SUB)])
    else:
        br = c + jnp.where(s >= 8, 1, 0)
        pltpu.sync_copy(bias_hbm.at[br], valb0.at[pl.ds(0, HD)])
        bv0 = valb0[pl.ds(0, 16)]
        bv1 = valb0[pl.ds(16, 16)]
        sec = ((c == 0) & (s >= 8)) | ((c == 1) & (s < 8))
        off2 = zbase + jnp.where(c == 0, -BPAD, BPAD)
        for p in range(ZP):
            k = p % KSUB
            pltpu.sync_copy(tbl_hbm.at[c, pl.ds(zbase + p * SUB, SUB)],
                            rowsb0.at[k])

            @pl.when(sec)
            def _(p=p, k=k):
                pltpu.sync_copy(tbl_hbm.at[1 - c, pl.ds(off2 + p * SUB, SUB)],
                                rowsb1.at[k])

                def abody(j, carry, k=k):
                    rowsb0[k, j, pl.ds(0, 16)] = (
                        rowsb0[k, j, pl.ds(0, 16)]
                        + rowsb1[k, j, pl.ds(0, 16)])
                    rowsb0[k, j, pl.ds(16, 16)] = (
                        rowsb0[k, j, pl.ds(16, 16)]
                        + rowsb1[k, j, pl.ds(16, 16)])
                    return carry

                lax.fori_loop(0, SUB, abody, 0)

            def bbody(j, carry, k=k):
                rowsb0[k, j, pl.ds(0, 16)] = rowsb0[k, j, pl.ds(0, 16)] + bv0
                rowsb0[k, j, pl.ds(16, 16)] = (
                    rowsb0[k, j, pl.ds(16, 16)] + bv1)
                return carry

            lax.fori_loop(0, SUB, bbody, 0)
            pltpu.sync_copy(rowsb0.at[k],
                            tbl_sh.at[pl.ds(zbase + p * SUB, SUB)])
    plsc.subcore_barrier()

    ebase = wid * PERW
    rbase = wid * (PERW // SUB)
    idxs = [(srcb0, dstb0, valb0, semld0),
            (srcb1, dstb1, valb1, semld1),
            (srcb2, dstb2, valb2, semld2)]
    rows = [(rowsb0, semg0, sems0), (rowsb1, semg1, sems1)]

    def fire_loads(g, iset):
        srcb, dstb, valb, semld = iset
        r0 = rbase + g * KSUB
        pltpu.async_copy(src_hbm.at[pl.ds(r0, KSUB)], srcb, semld)
        pltpu.async_copy(dst_hbm.at[pl.ds(r0, KSUB)], dstb, semld)
        pltpu.async_copy(val_hbm.at[pl.ds(ebase + g * CHUNK, CHUNK)],
                         valb, semld)

    def wait_loads(g, iset):
        srcb, dstb, valb, semld = iset
        r0 = rbase + g * KSUB
        pltpu.make_async_copy(src_hbm.at[pl.ds(r0, KSUB)], srcb, semld).wait()
        pltpu.make_async_copy(dst_hbm.at[pl.ds(r0, KSUB)], dstb, semld).wait()
        pltpu.make_async_copy(val_hbm.at[pl.ds(ebase + g * CHUNK, CHUNK)],
                              valb, semld).wait()

    def fire_gathers(iset, rset):
        srcb = iset[0]
        rowsb, semg, _ = rset
        for k in range(KSUB):
            pltpu.async_copy(tbl_sh.at[srcb.at[k]], rowsb.at[k], semg)

    def wait_gathers(iset, rset):
        srcb = iset[0]
        rowsb, semg, _ = rset
        for k in range(KSUB):
            pltpu.make_async_copy(tbl_sh.at[srcb.at[k]], rowsb.at[k],
                                  semg).wait()

    def scale_scatter(iset, rset):
        _, dstb, valb, _ = iset
        rowsb, _, sems = rset
        for k in range(KSUB):
            def mbody(g, mcarry, k=k):
                base = g * 16
                valv = valb[pl.ds(k * SUB + base, 16)]
                for l in range(16):
                    sv = valv[l]
                    e = base + l
                    rowsb[k, e, pl.ds(0, 16)] = rowsb[k, e, pl.ds(0, 16)] * sv
                    rowsb[k, e, pl.ds(16, 16)] = (
                        rowsb[k, e, pl.ds(16, 16)] * sv)
                return mcarry

            lax.fori_loop(0, SUB // 16, mbody, 0)
            pltpu.async_copy(rowsb.at[k], acc.at[dstb.at[k]], sems, add=True)

    def wait_scatters(iset, rset):
        dstb = iset[1]
        rowsb, _, sems = rset
        for k in range(KSUB):
            pltpu.make_async_copy(rowsb.at[k], acc.at[dstb.at[k]],
                                  sems).wait()

    def half(e, j):
        iset = idxs[j % 3]
        rset = rows[j % 2]
        wait_gathers(iset, rset)

        @pl.when((e >= 1) & (e + 1 < NCH))
        def _():
            wait_scatters(idxs[(j - 1) % 3], rows[(j + 1) % 2])

        @pl.when(e + 1 < NCH)
        def _():
            wait_loads(e + 1, idxs[(j + 1) % 3])
            fire_gathers(idxs[(j + 1) % 3], rows[(j + 1) % 2])

        scale_scatter(iset, rset)

        @pl.when(e + 2 < NCH)
        def _():
            fire_loads(e + 2, idxs[(j + 2) % 3])

    srcb_p, dstb_p, valb_p, _ = idxs[0]
    pltpu.sync_copy(src_hbm.at[pl.ds(rbase, KSUB)], srcb_p)
    pltpu.sync_copy(dst_hbm.at[pl.ds(rbase, KSUB)], dstb_p)
    pltpu.sync_copy(val_hbm.at[pl.ds(ebase, CHUNK)], valb_p)
    fire_gathers(idxs[0], rows[0])
    fire_loads(1, idxs[1])

    def pipe6(gg, carry):
        e0 = gg * 6
        for j in range(6):
            half(e0 + j, j)
        return carry

    lax.fori_loop(0, NCH // 6, pipe6, 0)
    wait_scatters(idxs[(NCH - 2) % 3], rows[(NCH - 2) % 2])
    wait_scatters(idxs[(NCH - 1) % 3], rows[(NCH - 1) % 2])
    plsc.subcore_barrier()

    for p in range(ZP):
        pltpu.sync_copy(acc.at[pl.ds(zbase + p * SUB, SUB)],
                        rowsb0.at[p % KSUB])
        pltpu.sync_copy(rowsb0.at[p % KSUB],
                        out_hbm.at[c, pl.ds(zbase + p * SUB, SUB)])


_SC_SCRATCH = [
    pltpu.VMEM((KSUB, SUB), _i32),
    pltpu.VMEM((KSUB, SUB), _i32),
    pltpu.VMEM((CHUNK,), _f32),
    pltpu.VMEM((KSUB, SUB), _i32),
    pltpu.VMEM((KSUB, SUB), _i32),
    pltpu.VMEM((CHUNK,), _f32),
    pltpu.VMEM((KSUB, SUB), _i32),
    pltpu.VMEM((KSUB, SUB), _i32),
    pltpu.VMEM((CHUNK,), _f32),
    pltpu.VMEM((KSUB, SUB, HD), _f32),
    pltpu.VMEM((KSUB, SUB, HD), _f32),
    pltpu.VMEM_SHARED((WROWS, HD), _f32),
    pltpu.VMEM_SHARED((WROWS, HD), _f32),
    pltpu.SemaphoreType.DMA,
    pltpu.SemaphoreType.DMA,
    pltpu.SemaphoreType.DMA,
    pltpu.SemaphoreType.DMA,
    pltpu.SemaphoreType.DMA,
    pltpu.SemaphoreType.DMA,
    pltpu.SemaphoreType.DMA,
]


def _spmm(src, dst, vals, tbl):
    def body(src_hbm, dst_hbm, val_hbm, tbl_hbm, out_hbm, *scr):
        _spmm_kernel_body(False, src_hbm, dst_hbm, val_hbm, tbl_hbm, None,
                          out_hbm, *scr)

    call = pl.kernel(
        body,
        out_type=jax.ShapeDtypeStruct((2, WROWS, HD), _f32),
        mesh=plsc.VectorSubcoreMesh(core_axis_name="c", subcore_axis_name="s"),
        scratch_types=_SC_SCRATCH,
        compiler_params=pltpu.CompilerParams(use_tc_tiling_on_sc=False),
    )
    return call(src, dst, vals, tbl)


def _spmm_rc(src, dst, vals, parts, bias):
    def body(src_hbm, dst_hbm, val_hbm, part_hbm, bias_hbm, out_hbm, *scr):
        _spmm_kernel_body(True, src_hbm, dst_hbm, val_hbm, part_hbm, bias_hbm,
                          out_hbm, *scr)

    call = pl.kernel(
        body,
        out_type=jax.ShapeDtypeStruct((2, WROWS, HD), _f32),
        mesh=plsc.VectorSubcoreMesh(core_axis_name="c", subcore_axis_name="s"),
        scratch_types=_SC_SCRATCH,
        compiler_params=pltpu.CompilerParams(use_tc_tiling_on_sc=False),
    )
    return call(src, dst, vals, parts, bias)


def _proj_body(x_ref, w_ref, o_ref):
    x = x_ref[0]
    y = jnp.where(x >= 0, x, 0.01 * x)
    o_ref[...] = lax.dot(y, w_ref[0], precision=lax.Precision.HIGHEST,
                         preferred_element_type=_f32)


_proj = pl.pallas_call(
    _proj_body,
    grid=(NBR,),
    in_specs=[
        pl.BlockSpec((1, BPAD, FF), lambda i: (i, 0, 0)),
        pl.BlockSpec((1, FF, HD), lambda i: (i, 0, 0)),
    ],
    out_specs=pl.BlockSpec((BPAD, HD), lambda i: (i, 0)),
    out_shape=jax.ShapeDtypeStruct((TPAD, HD), _f32),
)


def _mid_body(pa_ref, pb_ref, b_ref, w_ref, o_ref):
    i = pl.program_id(0)
    w0 = jnp.where(i <= 1, 1.0, 0.0)
    w1 = jnp.where(i >= 1, 1.0, 0.0)
    h = w0 * pa_ref[0] + w1 * pb_ref[0] + b_ref[0, 0]
    h = jnp.where(h >= 0, h, 0.01 * h)
    o_ref[...] = lax.dot(h, w_ref[0], precision=lax.Precision.HIGHEST,
                         preferred_element_type=_f32)


_mid_act = pl.pallas_call(
    _mid_body,
    grid=(NBR,),
    in_specs=[
        pl.BlockSpec((1, BPAD, HD), lambda i: (0, jnp.minimum(i, 1), 0)),
        pl.BlockSpec((1, BPAD, HD), lambda i: (1, jnp.maximum(i - 1, 0), 0)),
        pl.BlockSpec((1, 1, HD), lambda i: (i, 0, 0)),
        pl.BlockSpec((1, HD, HD), lambda i: (i, 0, 0)),
    ],
    out_specs=pl.BlockSpec((BPAD, HD), lambda i: (i, 0)),
    out_shape=jax.ShapeDtypeStruct((TPAD, HD), _f32),
)


def _pool_body(p_ref, b3_ref, bt_ref, w3_ref, wout_ref, bout_ref, o_ref):
    tot = jnp.zeros((GG, HD), _f32)
    for i in range(NBR):
        if i == 0:
            t = p_ref[0, pl.ds(0, BPAD)]
        elif i == 1:
            t = p_ref[0, pl.ds(BPAD, BPAD)] + p_ref[1, pl.ds(0, BPAD)]
        else:
            t = p_ref[1, pl.ds(BPAD, BPAD)]
        bt = bt_ref[i, 0]
        oh = (bt[None, :] == lax.broadcasted_iota(_i32, (GG, BPAD), 0)
              ).astype(_f32)
        ssum = lax.dot(oh, t, precision=lax.Precision.HIGHEST,
                       preferred_element_type=_f32)
        ssum = lax.dot(ssum, w3_ref[i], precision=lax.Precision.HIGHEST,
                       preferred_element_type=_f32)
        cnt = jnp.sum(oh, axis=1)
        gm = ssum + cnt[:, None] * b3_ref[i, 0][None, :]
        tot = tot + gm / jnp.maximum(cnt, 1.0)[:, None]
    logits = lax.dot(tot, wout_ref[...], precision=lax.Precision.HIGHEST,
                     preferred_element_type=_f32) + bout_ref[0][None, :]
    m = jnp.max(logits, axis=0, keepdims=True)
    ex = jnp.exp(logits - m)
    o_ref[...] = ex / jnp.sum(ex, axis=0, keepdims=True)


_pool = pl.pallas_call(
    _pool_body,
    out_shape=jax.ShapeDtypeStruct((GG, OUTD), _f32),
)


def kernel(X0, X1, X2, L0_idx, L0_val, L1_idx, L1_val, L2_idx, L2_val,
           batch0, batch1, batch2,
           W01, b01, W02, b02, W03, b03,
           W11, b11, W12, b12, W13, b13,
           W21, b21, W22, b22, W23, b23,
           Wout, bout):
    pad = EPAD - NBR * EE
    b1s_split = CORE0E - EE
    src1 = L1_idx[1].astype(_i32)
    dst1 = L1_idx[0].astype(_i32)
    src = jnp.concatenate([
        L0_idx[1].astype(_i32), src1[:b1s_split] + BPAD, src1[b1s_split:],
        L2_idx[1].astype(_i32) + BPAD, jnp.zeros((pad,), _i32),
    ]).reshape(EROWS, SUB)
    dst = jnp.concatenate([
        L0_idx[0].astype(_i32), dst1[:b1s_split] + BPAD, dst1[b1s_split:],
        L2_idx[0].astype(_i32) + BPAD, jnp.zeros((pad,), _i32),
    ]).reshape(EROWS, SUB)
    vals = jnp.concatenate([L0_val, L1_val, L2_val, jnp.zeros((pad,), _f32)])

    Xs = jnp.pad(jnp.stack([X0, X1, X2]), ((0, 0), (0, BPAD - NN), (0, 0)))
    W1s = jnp.stack([W01, W11, W21])
    W2s = jnp.stack([W02, W12, W22])
    W3s = jnp.stack([W03, W13, W23])
    b1s = jnp.stack([b01, b11, b21]).reshape(NBR, 1, HD)
    b2s = jnp.stack([b02, b12, b22])
    b3s = jnp.stack([b03, b13, b23]).reshape(NBR, 1, HD)
    bts = jnp.pad(jnp.stack([batch0, batch1, batch2]).astype(_i32),
                  ((0, 0), (0, BPAD - NN)),
                  constant_values=-1).reshape(NBR, 1, BPAD)

    y = _proj(Xs, W1s)
    p1 = _spmm(src, dst, vals, y)
    h1 = _mid_act(p1, p1, b1s, W2s)
    p2 = _spmm(src, dst, vals, h1)
    p3 = _spmm_rc(src, dst, vals, p2, b2s)
    return _pool(p3, b3s, bts, W3s, Wout, bout.reshape(1, OUTD))


# speedup vs baseline: 1.0010x; 1.0010x over previous
import functools

import jax
import jax.numpy as jnp
from jax import lax
from jax.experimental import pallas as pl
from jax.experimental.pallas import tpu as pltpu
from jax.experimental.pallas import tpu_sc as plsc

NN = 10000
EE = 320000
GG = 64
FF = 128
HD = 32
OUTD = 10

NBR = 3
BPAD = 10240
TPAD = NBR * BPAD
WROWS = 2 * BPAD

NWORK = 32
SUB = 128
KSUB = 4
CHUNK = SUB * KSUB
EPAD = 983040
PERW = EPAD // NWORK
NCH = PERW // CHUNK
EROWS = EPAD // SUB
CORE0E = 16 * PERW
ZROWS = WROWS // 16
ZP = ZROWS // SUB

_f32 = jnp.float32
_i32 = jnp.int32


def _spmm_kernel_body(recon, src_hbm, dst_hbm, val_hbm, tbl_hbm, bias_hbm,
                      out_hbm, srcb0, dstb0, valb0, srcb1, dstb1, valb1,
                      srcb2, dstb2, valb2, rowsb0, rowsb1, tbl_sh, acc,
                      semld0, semld1, semld2, semg0, semg1, sems0, sems1):
    c = lax.axis_index("c")
    s = lax.axis_index("s")
    wid = c * 16 + s
    zbase = s * ZROWS

    zero16 = jnp.zeros((16,), _f32)
    for k in range(KSUB):
        def zbody(j, carry, k=k):
            rowsb0[k, j, pl.ds(0, 16)] = zero16
            rowsb0[k, j, pl.ds(16, 16)] = zero16
            return carry

        lax.fori_loop(0, SUB, zbody, 0)
    for p in range(ZP):
        pltpu.sync_copy(rowsb0.at[p % KSUB],
                        acc.at[pl.ds(zbase + p * SUB, SUB)])

    if not recon:
        wbase = c * BPAD
        for p in range(ZP):
            pltpu.sync_copy(tbl_hbm.at[pl.ds(wbase + zbase + p * SUB, SUB)],
                            rowsb1.at[p % KSUB])
            pltpu.sync_copy(rowsb1.at[p % KSUB],
                            tbl_sh.at[pl.ds(zbase + p * SUB, SUB)])
    else:
        br = c + jnp.where(s >= 8, 1, 0)
        pltpu.sync_copy(bias_hbm.at[br], valb0.at[pl.ds(0, HD)])
        bv0 = valb0[pl.ds(0, 16)]
        bv1 = valb0[pl.ds(16, 16)]
        sec = ((c == 0) & (s >= 8)) | ((c == 1) & (s < 8))
        off2 = zbase + jnp.where(c == 0, -BPAD, BPAD)
        for p in range(ZP):
            k = p % KSUB
            pltpu.sync_copy(tbl_hbm.at[c, pl.ds(zbase + p * SUB, SUB)],
                            rowsb0.at[k])

            @pl.when(sec)
            def _(p=p, k=k):
                pltpu.sync_copy(tbl_hbm.at[1 - c, pl.ds(off2 + p * SUB, SUB)],
                                rowsb1.at[k])

                def abody(j, carry, k=k):
                    rowsb0[k, j, pl.ds(0, 16)] = (
                        rowsb0[k, j, pl.ds(0, 16)]
                        + rowsb1[k, j, pl.ds(0, 16)])
                    rowsb0[k, j, pl.ds(16, 16)] = (
                        rowsb0[k, j, pl.ds(16, 16)]
                        + rowsb1[k, j, pl.ds(16, 16)])
                    return carry

                lax.fori_loop(0, SUB, abody, 0)

            def bbody(j, carry, k=k):
                rowsb0[k, j, pl.ds(0, 16)] = rowsb0[k, j, pl.ds(0, 16)] + bv0
                rowsb0[k, j, pl.ds(16, 16)] = (
                    rowsb0[k, j, pl.ds(16, 16)] + bv1)
                return carry

            lax.fori_loop(0, SUB, bbody, 0)
            pltpu.sync_copy(rowsb0.at[k],
                            tbl_sh.at[pl.ds(zbase + p * SUB, SUB)])
    plsc.subcore_barrier()

    ebase = wid * PERW
    rbase = wid * (PERW // SUB)
    idxs = [(srcb0, dstb0, valb0, semld0),
            (srcb1, dstb1, valb1, semld1),
            (srcb2, dstb2, valb2, semld2)]
    rows = [(rowsb0, semg0, sems0), (rowsb1, semg1, sems1)]

    def fire_loads(g, iset):
        srcb, dstb, valb, semld = iset
        r0 = rbase + g * KSUB
        pltpu.async_copy(src_hbm.at[pl.ds(r0, KSUB)], srcb, semld)
        pltpu.async_copy(dst_hbm.at[pl.ds(r0, KSUB)], dstb, semld)
        pltpu.async_copy(val_hbm.at[pl.ds(ebase + g * CHUNK, CHUNK)],
                         valb, semld)

    def wait_loads(g, iset):
        srcb, dstb, valb, semld = iset
        r0 = rbase + g * KSUB
        pltpu.make_async_copy(src_hbm.at[pl.ds(r0, KSUB)], srcb, semld).wait()
        pltpu.make_async_copy(dst_hbm.at[pl.ds(r0, KSUB)], dstb, semld).wait()
        pltpu.make_async_copy(val_hbm.at[pl.ds(ebase + g * CHUNK, CHUNK)],
                              valb, semld).wait()

    def fire_gathers(iset, rset):
        srcb = iset[0]
        rowsb, semg, _ = rset
        for k in range(KSUB):
            pltpu.async_copy(tbl_sh.at[srcb.at[k]], rowsb.at[k], semg)

    def wait_gathers(iset, rset):
        srcb = iset[0]
        rowsb, semg, _ = rset
        for k in range(KSUB):
            pltpu.make_async_copy(tbl_sh.at[srcb.at[k]], rowsb.at[k],
                                  semg).wait()

    def scale_scatter(iset, rset):
        _, dstb, valb, _ = iset
        rowsb, _, sems = rset
        for k in range(KSUB):
            def mbody(g, mcarry, k=k):
                base = g * 16
                valv = valb[pl.ds(k * SUB + base, 16)]
                for l in range(16):
                    sv = valv[l]
                    e = base + l
                    rowsb[k, e, pl.ds(0, 16)] = rowsb[k, e, pl.ds(0, 16)] * sv
                    rowsb[k, e, pl.ds(16, 16)] = (
                        rowsb[k, e, pl.ds(16, 16)] * sv)
                return mcarry

            lax.fori_loop(0, SUB // 16, mbody, 0)
            pltpu.async_copy(rowsb.at[k], acc.at[dstb.at[k]], sems, add=True)

    def wait_scatters(iset, rset):
        dstb = iset[1]
        rowsb, _, sems = rset
        for k in range(KSUB):
            pltpu.make_async_copy(rowsb.at[k], acc.at[dstb.at[k]],
                                  sems).wait()

    def half(e, j):
        iset = idxs[j % 3]
        rset = rows[j % 2]
        wait_gathers(iset, rset)

        @pl.when((e >= 1) & (e + 1 < NCH))
        def _():
            wait_scatters(idxs[(j - 1) % 3], rows[(j + 1) % 2])

        @pl.when(e + 1 < NCH)
        def _():
            wait_loads(e + 1, idxs[(j + 1) % 3])
            fire_gathers(idxs[(j + 1) % 3], rows[(j + 1) % 2])

        scale_scatter(iset, rset)

        @pl.when(e + 2 < NCH)
        def _():
            fire_loads(e + 2, idxs[(j + 2) % 3])

    srcb_p, dstb_p, valb_p, _ = idxs[0]
    pltpu.sync_copy(src_hbm.at[pl.ds(rbase, KSUB)], srcb_p)
    pltpu.sync_copy(dst_hbm.at[pl.ds(rbase, KSUB)], dstb_p)
    pltpu.sync_copy(val_hbm.at[pl.ds(ebase, CHUNK)], valb_p)
    fire_gathers(idxs[0], rows[0])
    fire_loads(1, idxs[1])

    def pipe6(gg, carry):
        e0 = gg * 6
        for j in range(6):
            half(e0 + j, j)
        return carry

    lax.fori_loop(0, NCH // 6, pipe6, 0)
    wait_scatters(idxs[(NCH - 2) % 3], rows[(NCH - 2) % 2])
    wait_scatters(idxs[(NCH - 1) % 3], rows[(NCH - 1) % 2])
    plsc.subcore_barrier()

    for p in range(ZP):
        pltpu.sync_copy(acc.at[pl.ds(zbase + p * SUB, SUB)],
                        rowsb0.at[p % KSUB])
        pltpu.sync_copy(rowsb0.at[p % KSUB],
                        out_hbm.at[c, pl.ds(zbase + p * SUB, SUB)])


_SC_SCRATCH = [
    pltpu.VMEM((KSUB, SUB), _i32),
    pltpu.VMEM((KSUB, SUB), _i32),
    pltpu.VMEM((CHUNK,), _f32),
    pltpu.VMEM((KSUB, SUB), _i32),
    pltpu.VMEM((KSUB, SUB), _i32),
    pltpu.VMEM((CHUNK,), _f32),
    pltpu.VMEM((KSUB, SUB), _i32),
    pltpu.VMEM((KSUB, SUB), _i32),
    pltpu.VMEM((CHUNK,), _f32),
    pltpu.VMEM((KSUB, SUB, HD), _f32),
    pltpu.VMEM((KSUB, SUB, HD), _f32),
    pltpu.VMEM_SHARED((WROWS, HD), _f32),
    pltpu.VMEM_SHARED((WROWS, HD), _f32),
    pltpu.SemaphoreType.DMA,
    pltpu.SemaphoreType.DMA,
    pltpu.SemaphoreType.DMA,
    pltpu.SemaphoreType.DMA,
    pltpu.SemaphoreType.DMA,
    pltpu.SemaphoreType.DMA,
    pltpu.SemaphoreType.DMA,
]


def _spmm(src, dst, vals, tbl):
    def body(src_hbm, dst_hbm, val_hbm, tbl_hbm, out_hbm, *scr):
        _spmm_kernel_body(False, src_hbm, dst_hbm, val_hbm, tbl_hbm, None,
                          out_hbm, *scr)

    call = pl.kernel(
        body,
        out_type=jax.ShapeDtypeStruct((2, WROWS, HD), _f32),
        mesh=plsc.VectorSubcoreMesh(core_axis_name="c", subcore_axis_name="s"),
        scratch_types=_SC_SCRATCH,
        compiler_params=pltpu.CompilerParams(use_tc_tiling_on_sc=False),
    )
    return call(src, dst, vals, tbl)


def _spmm_rc(src, dst, vals, parts, bias):
    def body(src_hbm, dst_hbm, val_hbm, part_hbm, bias_hbm, out_hbm, *scr):
        _spmm_kernel_body(True, src_hbm, dst_hbm, val_hbm, part_hbm, bias_hbm,
                          out_hbm, *scr)

    call = pl.kernel(
        body,
        out_type=jax.ShapeDtypeStruct((2, WROWS, HD), _f32),
        mesh=plsc.VectorSubcoreMesh(core_axis_name="c", subcore_axis_name="s"),
        scratch_types=_SC_SCRATCH,
        compiler_params=pltpu.CompilerParams(use_tc_tiling_on_sc=False),
    )
    return call(src, dst, vals, parts, bias)


def _proj_body(x_ref, w_ref, o_ref):
    x = x_ref[0]
    y = jnp.where(x >= 0, x, 0.01 * x)
    o_ref[0] = lax.dot(y, w_ref[0], precision=lax.Precision.HIGHEST,
                       preferred_element_type=_f32)


_proj = pl.pallas_call(
    _proj_body,
    grid=(NBR,),
    in_specs=[
        pl.BlockSpec((1, BPAD, FF), lambda i: (i, 0, 0)),
        pl.BlockSpec((1, FF, HD), lambda i: (i, 0, 0)),
    ],
    out_specs=pl.BlockSpec((1, BPAD, HD), lambda i: (i, 0, 0)),
    out_shape=jax.ShapeDtypeStruct((NBR, BPAD, HD), _f32),
)


def _mid_body(pa_ref, pb_ref, b_ref, w_ref, o_ref):
    i = pl.program_id(0)
    w0 = jnp.where(i <= 1, 1.0, 0.0)
    w1 = jnp.where(i >= 1, 1.0, 0.0)
    h = w0 * pa_ref[0] + w1 * pb_ref[0] + b_ref[0, 0]
    h = jnp.where(h >= 0, h, 0.01 * h)
    o_ref[0] = lax.dot(h, w_ref[0], precision=lax.Precision.HIGHEST,
                       preferred_element_type=_f32)


_mid_act = pl.pallas_call(
    _mid_body,
    grid=(NBR,),
    in_specs=[
        pl.BlockSpec((1, BPAD, HD), lambda i: (0, jnp.minimum(i, 1), 0)),
        pl.BlockSpec((1, BPAD, HD), lambda i: (1, jnp.maximum(i - 1, 0), 0)),
        pl.BlockSpec((1, 1, HD), lambda i: (i, 0, 0)),
        pl.BlockSpec((1, HD, HD), lambda i: (i, 0, 0)),
    ],
    out_specs=pl.BlockSpec((1, BPAD, HD), lambda i: (i, 0, 0)),
    out_shape=jax.ShapeDtypeStruct((NBR, BPAD, HD), _f32),
)


def _pool_body(p_ref, b3_ref, bt_ref, w3_ref, wout_ref, bout_ref, o_ref):
    tot = jnp.zeros((GG, HD), _f32)
    for i in range(NBR):
        if i == 0:
            t = p_ref[0, pl.ds(0, BPAD)]
        elif i == 1:
            t = p_ref[0, pl.ds(BPAD, BPAD)] + p_ref[1, pl.ds(0, BPAD)]
        else:
            t = p_ref[1, pl.ds(BPAD, BPAD)]
        bt = bt_ref[i, 0]
        oh = (bt[None, :] == lax.broadcasted_iota(_i32, (GG, BPAD), 0)
              ).astype(_f32)
        ssum = lax.dot(oh, t, precision=lax.Precision.HIGHEST,
                       preferred_element_type=_f32)
        ssum = lax.dot(ssum, w3_ref[i], precision=lax.Precision.HIGHEST,
                       preferred_element_type=_f32)
        cnt = jnp.sum(oh, axis=1)
        gm = ssum + cnt[:, None] * b3_ref[i, 0][None, :]
        tot = tot + gm / jnp.maximum(cnt, 1.0)[:, None]
    logits = lax.dot(tot, wout_ref[...], precision=lax.Precision.HIGHEST,
                     preferred_element_type=_f32) + bout_ref[0][None, :]
    m = jnp.max(logits, axis=0, keepdims=True)
    ex = jnp.exp(logits - m)
    o_ref[...] = ex / jnp.sum(ex, axis=0, keepdims=True)


_pool = pl.pallas_call(
    _pool_body,
    out_shape=jax.ShapeDtypeStruct((GG, OUTD), _f32),
)


def kernel(X0, X1, X2, L0_idx, L0_val, L1_idx, L1_val, L2_idx, L2_val,
           batch0, batch1, batch2,
           W01, b01, W02, b02, W03, b03,
           W11, b11, W12, b12, W13, b13,
           W21, b21, W22, b22, W23, b23,
           Wout, bout):
    pad = EPAD - NBR * EE
    b1s_split = CORE0E - EE
    src1 = L1_idx[1].astype(_i32)
    dst1 = L1_idx[0].astype(_i32)
    src = jnp.concatenate([
        L0_idx[1].astype(_i32), src1[:b1s_split] + BPAD, src1[b1s_split:],
        L2_idx[1].astype(_i32) + BPAD, jnp.zeros((pad,), _i32),
    ]).reshape(EROWS, SUB)
    dst = jnp.concatenate([
        L0_idx[0].astype(_i32), dst1[:b1s_split] + BPAD, dst1[b1s_split:],
        L2_idx[0].astype(_i32) + BPAD, jnp.zeros((pad,), _i32),
    ]).reshape(EROWS, SUB)
    vals = jnp.concatenate([L0_val, L1_val, L2_val, jnp.zeros((pad,), _f32)])

    Xs = jnp.pad(jnp.stack([X0, X1, X2]), ((0, 0), (0, BPAD - NN), (0, 0)))
    W1s = jnp.stack([W01, W11, W21])
    W2s = jnp.stack([W02, W12, W22])
    W3s = jnp.stack([W03, W13, W23])
    b1s = jnp.stack([b01, b11, b21]).reshape(NBR, 1, HD)
    b2s = jnp.stack([b02, b12, b22])
    b3s = jnp.stack([b03, b13, b23]).reshape(NBR, 1, HD)
    bts = jnp.pad(jnp.stack([batch0, batch1, batch2]).astype(_i32),
                  ((0, 0), (0, BPAD - NN)),
                  constant_values=-1).reshape(NBR, 1, BPAD)

    y = _proj(Xs, W1s)
    p1 = _spmm(src, dst, vals, y.reshape(TPAD, HD))
    h1 = _mid_act(p1, p1, b1s, W2s)
    p2 = _spmm(src, dst, vals, h1.reshape(TPAD, HD))
    p3 = _spmm_rc(src, dst, vals, p2, b2s)
    return _pool(p3, b3s, bts, W3s, Wout, bout.reshape(1, OUTD))


# speedup vs baseline: 1.0992x; 1.0981x over previous
import functools

import jax
import jax.numpy as jnp
from jax import lax
from jax.experimental import pallas as pl
from jax.experimental.pallas import tpu as pltpu
from jax.experimental.pallas import tpu_sc as plsc

NN = 10000
EE = 320000
GG = 64
FF = 128
HD = 32
OUTD = 10

NBR = 3
BPAD = 10240
TPAD = NBR * BPAD
WROWS = 2 * BPAD

NWORK = 32
SUB = 128
KSUB = 4
CHUNK = SUB * KSUB
EPAD = 983040
PERW = EPAD // NWORK
NCH = PERW // CHUNK
EROWS = EPAD // SUB
CORE0E = 16 * PERW
ZROWS = WROWS // 16
ZP = ZROWS // SUB

_f32 = jnp.float32
_i32 = jnp.int32


def _spmm_kernel_body(recon, src_hbm, dst_hbm, val_hbm, tbl_hbm, bias_hbm,
                      out_hbm, srcb0, dstb0, valb0, srcb1, dstb1, valb1,
                      srcb2, dstb2, valb2, rowsb0, rowsb1, tbl_sh, acc,
                      semld0, semld1, semld2, semg0, semg1, sems0, sems1):
    c = lax.axis_index("c")
    s = lax.axis_index("s")
    wid = c * 16 + s
    zbase = s * ZROWS

    zero16 = jnp.zeros((16,), _f32)
    for k in range(KSUB):
        def zbody(j, carry, k=k):
            rowsb0[k, j, pl.ds(0, 16)] = zero16
            rowsb0[k, j, pl.ds(16, 16)] = zero16
            return carry

        lax.fori_loop(0, SUB, zbody, 0)
    for p in range(ZP):
        pltpu.sync_copy(rowsb0.at[p % KSUB],
                        acc.at[pl.ds(zbase + p * SUB, SUB)])

    if not recon:
        wbase = c * BPAD
        for p in range(ZP):
            pltpu.sync_copy(tbl_hbm.at[pl.ds(wbase + zbase + p * SUB, SUB)],
                            rowsb1.at[p % KSUB])
            pltpu.sync_copy(rowsb1.at[p % KSUB],
                            tbl_sh.at[pl.ds(zbase + p * SUB, SUB)])
    else:
        br = c + jnp.where(s >= 8, 1, 0)
        pltpu.sync_copy(bias_hbm.at[br], valb0.at[pl.ds(0, HD)])
        bv0 = valb0[pl.ds(0, 16)]
        bv1 = valb0[pl.ds(16, 16)]
        sec = ((c == 0) & (s >= 8)) | ((c == 1) & (s < 8))
        off2 = zbase + jnp.where(c == 0, -BPAD, BPAD)
        for p in range(ZP):
            k = p % KSUB
            pltpu.sync_copy(tbl_hbm.at[c, pl.ds(zbase + p * SUB, SUB)],
                            rowsb0.at[k])

            @pl.when(sec)
            def _(p=p, k=k):
                pltpu.sync_copy(tbl_hbm.at[1 - c, pl.ds(off2 + p * SUB, SUB)],
                                rowsb1.at[k])

                def abody(j, carry, k=k):
                    rowsb0[k, j, pl.ds(0, 16)] = (
                        rowsb0[k, j, pl.ds(0, 16)]
                        + rowsb1[k, j, pl.ds(0, 16)])
                    rowsb0[k, j, pl.ds(16, 16)] = (
                        rowsb0[k, j, pl.ds(16, 16)]
                        + rowsb1[k, j, pl.ds(16, 16)])
                    return carry

                lax.fori_loop(0, SUB, abody, 0)

            def bbody(j, carry, k=k):
                rowsb0[k, j, pl.ds(0, 16)] = rowsb0[k, j, pl.ds(0, 16)] + bv0
                rowsb0[k, j, pl.ds(16, 16)] = (
                    rowsb0[k, j, pl.ds(16, 16)] + bv1)
                return carry

            lax.fori_loop(0, SUB, bbody, 0)
            pltpu.sync_copy(rowsb0.at[k],
                            tbl_sh.at[pl.ds(zbase + p * SUB, SUB)])
    plsc.subcore_barrier()

    ebase = wid * PERW
    rbase = wid * (PERW // SUB)
    idxs = [(srcb0, dstb0, valb0, semld0),
            (srcb1, dstb1, valb1, semld1),
            (srcb2, dstb2, valb2, semld2)]
    rows = [(rowsb0, semg0, sems0), (rowsb1, semg1, sems1)]

    def fire_loads(g, iset):
        srcb, dstb, valb, semld = iset
        r0 = rbase + g * KSUB
        pltpu.async_copy(src_hbm.at[pl.ds(r0, KSUB)], srcb, semld)
        pltpu.async_copy(dst_hbm.at[pl.ds(r0, KSUB)], dstb, semld)
        pltpu.async_copy(val_hbm.at[pl.ds(ebase + g * CHUNK, CHUNK)],
                         valb, semld)

    def wait_loads(g, iset):
        srcb, dstb, valb, semld = iset
        r0 = rbase + g * KSUB
        pltpu.make_async_copy(src_hbm.at[pl.ds(r0, KSUB)], srcb, semld).wait()
        pltpu.make_async_copy(dst_hbm.at[pl.ds(r0, KSUB)], dstb, semld).wait()
        pltpu.make_async_copy(val_hbm.at[pl.ds(ebase + g * CHUNK, CHUNK)],
                              valb, semld).wait()

    def fire_gathers(iset, rset):
        srcb = iset[0]
        rowsb, semg, _ = rset
        for k in range(KSUB):
            pltpu.async_copy(tbl_sh.at[srcb.at[k]], rowsb.at[k], semg)

    def wait_gathers(iset, rset):
        srcb = iset[0]
        rowsb, semg, _ = rset
        for k in range(KSUB):
            pltpu.make_async_copy(tbl_sh.at[srcb.at[k]], rowsb.at[k],
                                  semg).wait()

    def scale_scatter(iset, rset):
        _, dstb, valb, _ = iset
        rowsb, _, sems = rset
        for k in range(KSUB):
            def mbody(g, mcarry, k=k):
                base = g * 16
                valv = valb[pl.ds(k * SUB + base, 16)]
                for l in range(16):
                    sv = valv[l]
                    e = base + l
                    rowsb[k, e, pl.ds(0, 16)] = rowsb[k, e, pl.ds(0, 16)] * sv
                    rowsb[k, e, pl.ds(16, 16)] = (
                        rowsb[k, e, pl.ds(16, 16)] * sv)
                return mcarry

            lax.fori_loop(0, SUB // 16, mbody, 0)
            pltpu.async_copy(rowsb.at[k], acc.at[dstb.at[k]], sems, add=True)

    def wait_scatters(iset, rset):
        dstb = iset[1]
        rowsb, _, sems = rset
        for k in range(KSUB):
            pltpu.make_async_copy(rowsb.at[k], acc.at[dstb.at[k]],
                                  sems).wait()

    def half(e, j):
        iset = idxs[j % 3]
        rset = rows[j % 2]
        wait_gathers(iset, rset)

        @pl.when((e >= 1) & (e + 1 < NCH))
        def _():
            wait_scatters(idxs[(j - 1) % 3], rows[(j + 1) % 2])

        @pl.when(e + 1 < NCH)
        def _():
            wait_loads(e + 1, idxs[(j + 1) % 3])
            fire_gathers(idxs[(j + 1) % 3], rows[(j + 1) % 2])

        scale_scatter(iset, rset)

        @pl.when(e + 2 < NCH)
        def _():
            fire_loads(e + 2, idxs[(j + 2) % 3])

    srcb_p, dstb_p, valb_p, _ = idxs[0]
    pltpu.sync_copy(src_hbm.at[pl.ds(rbase, KSUB)], srcb_p)
    pltpu.sync_copy(dst_hbm.at[pl.ds(rbase, KSUB)], dstb_p)
    pltpu.sync_copy(val_hbm.at[pl.ds(ebase, CHUNK)], valb_p)
    fire_gathers(idxs[0], rows[0])
    fire_loads(1, idxs[1])

    def pipe6(gg, carry):
        e0 = gg * 6
        for j in range(6):
            half(e0 + j, j)
        return carry

    lax.fori_loop(0, NCH // 6, pipe6, 0)
    wait_scatters(idxs[(NCH - 2) % 3], rows[(NCH - 2) % 2])
    wait_scatters(idxs[(NCH - 1) % 3], rows[(NCH - 1) % 2])
    plsc.subcore_barrier()

    for p in range(ZP):
        pltpu.sync_copy(acc.at[pl.ds(zbase + p * SUB, SUB)],
                        rowsb0.at[p % KSUB])
        pltpu.sync_copy(rowsb0.at[p % KSUB],
                        out_hbm.at[c, pl.ds(zbase + p * SUB, SUB)])


_SC_SCRATCH = [
    pltpu.VMEM((KSUB, SUB), _i32),
    pltpu.VMEM((KSUB, SUB), _i32),
    pltpu.VMEM((CHUNK,), _f32),
    pltpu.VMEM((KSUB, SUB), _i32),
    pltpu.VMEM((KSUB, SUB), _i32),
    pltpu.VMEM((CHUNK,), _f32),
    pltpu.VMEM((KSUB, SUB), _i32),
    pltpu.VMEM((KSUB, SUB), _i32),
    pltpu.VMEM((CHUNK,), _f32),
    pltpu.VMEM((KSUB, SUB, HD), _f32),
    pltpu.VMEM((KSUB, SUB, HD), _f32),
    pltpu.VMEM_SHARED((WROWS, HD), _f32),
    pltpu.VMEM_SHARED((WROWS, HD), _f32),
    pltpu.SemaphoreType.DMA,
    pltpu.SemaphoreType.DMA,
    pltpu.SemaphoreType.DMA,
    pltpu.SemaphoreType.DMA,
    pltpu.SemaphoreType.DMA,
    pltpu.SemaphoreType.DMA,
    pltpu.SemaphoreType.DMA,
]


def _spmm(src, dst, vals, tbl):
    def body(src_hbm, dst_hbm, val_hbm, tbl_hbm, out_hbm, *scr):
        _spmm_kernel_body(False, src_hbm, dst_hbm, val_hbm, tbl_hbm, None,
                          out_hbm, *scr)

    call = pl.kernel(
        body,
        out_type=jax.ShapeDtypeStruct((2, WROWS, HD), _f32),
        mesh=plsc.VectorSubcoreMesh(core_axis_name="c", subcore_axis_name="s"),
        scratch_types=_SC_SCRATCH,
        compiler_params=pltpu.CompilerParams(use_tc_tiling_on_sc=False),
    )
    return call(src, dst, vals, tbl)


def _spmm_rc(src, dst, vals, parts, bias):
    def body(src_hbm, dst_hbm, val_hbm, part_hbm, bias_hbm, out_hbm, *scr):
        _spmm_kernel_body(True, src_hbm, dst_hbm, val_hbm, part_hbm, bias_hbm,
                          out_hbm, *scr)

    call = pl.kernel(
        body,
        out_type=jax.ShapeDtypeStruct((2, WROWS, HD), _f32),
        mesh=plsc.VectorSubcoreMesh(core_axis_name="c", subcore_axis_name="s"),
        scratch_types=_SC_SCRATCH,
        compiler_params=pltpu.CompilerParams(use_tc_tiling_on_sc=False),
    )
    return call(src, dst, vals, parts, bias)


def _proj_body(x_ref, w_ref, o_ref):
    x = x_ref[0]
    y = jnp.where(x >= 0, x, 0.01 * x)
    o_ref[0] = lax.dot(y, w_ref[0], precision=lax.Precision.HIGHEST,
                       preferred_element_type=_f32)


_proj = pl.pallas_call(
    _proj_body,
    grid=(NBR,),
    in_specs=[
        pl.BlockSpec((1, BPAD, FF), lambda i: (i, 0, 0)),
        pl.BlockSpec((1, FF, HD), lambda i: (i, 0, 0)),
    ],
    out_specs=pl.BlockSpec((1, BPAD, HD), lambda i: (i, 0, 0)),
    out_shape=jax.ShapeDtypeStruct((NBR, BPAD, HD), _f32),
)


def _mid_body(pa_ref, pb_ref, b_ref, w_ref, o_ref):
    i = pl.program_id(0)
    w0 = jnp.where(i <= 1, 1.0, 0.0)
    w1 = jnp.where(i >= 1, 1.0, 0.0)
    h = w0 * pa_ref[0] + w1 * pb_ref[0] + b_ref[0, 0]
    h = jnp.where(h >= 0, h, 0.01 * h)
    o_ref[0] = lax.dot(h, w_ref[0], precision=lax.Precision.HIGHEST,
                       preferred_element_type=_f32)


_mid_act = pl.pallas_call(
    _mid_body,
    grid=(NBR,),
    in_specs=[
        pl.BlockSpec((1, BPAD, HD), lambda i: (0, jnp.minimum(i, 1), 0)),
        pl.BlockSpec((1, BPAD, HD), lambda i: (1, jnp.maximum(i - 1, 0), 0)),
        pl.BlockSpec((1, 1, HD), lambda i: (i, 0, 0)),
        pl.BlockSpec((1, HD, HD), lambda i: (i, 0, 0)),
    ],
    out_specs=pl.BlockSpec((1, BPAD, HD), lambda i: (i, 0, 0)),
    out_shape=jax.ShapeDtypeStruct((NBR, BPAD, HD), _f32),
)


def _pool_body(p_ref, b3_ref, bt_ref, w3_ref, wout_ref, bout_ref, o_ref):
    tot = jnp.zeros((GG, HD), _f32)
    for i in range(NBR):
        if i == 0:
            t = p_ref[0, pl.ds(0, BPAD)]
        elif i == 1:
            t = p_ref[0, pl.ds(BPAD, BPAD)] + p_ref[1, pl.ds(0, BPAD)]
        else:
            t = p_ref[1, pl.ds(BPAD, BPAD)]
        bt = bt_ref[i, 0]
        oh = (bt[None, :] == lax.broadcasted_iota(_i32, (GG, BPAD), 0)
              ).astype(_f32)
        ssum = lax.dot(oh, t, precision=lax.Precision.HIGHEST,
                       preferred_element_type=_f32)
        ssum = lax.dot(ssum, w3_ref[i], precision=lax.Precision.HIGHEST,
                       preferred_element_type=_f32)
        cnt = jnp.sum(oh, axis=1)
        gm = ssum + cnt[:, None] * b3_ref[i, 0][None, :]
        tot = tot + gm / jnp.maximum(cnt, 1.0)[:, None]
    logits = lax.dot(tot, wout_ref[...], precision=lax.Precision.HIGHEST,
                     preferred_element_type=_f32) + bout_ref[0][None, :]
    m = jnp.max(logits, axis=0, keepdims=True)
    ex = jnp.exp(logits - m)
    o_ref[...] = ex / jnp.sum(ex, axis=0, keepdims=True)


_pool = pl.pallas_call(
    _pool_body,
    out_shape=jax.ShapeDtypeStruct((GG, OUTD), _f32),
)


def kernel(X0, X1, X2, L0_idx, L0_val, L1_idx, L1_val, L2_idx, L2_val,
           batch0, batch1, batch2,
           W01, b01, W02, b02, W03, b03,
           W11, b11, W12, b12, W13, b13,
           W21, b21, W22, b22, W23, b23,
           Wout, bout):
    pad = EPAD - NBR * EE
    src = jnp.concatenate([
        L0_idx[1].astype(_i32), L1_idx[1].astype(_i32) + BPAD,
        L2_idx[1].astype(_i32) + 2 * BPAD, jnp.full((pad,), BPAD, _i32),
    ])
    dst = jnp.concatenate([
        L0_idx[0].astype(_i32), L1_idx[0].astype(_i32) + BPAD,
        L2_idx[0].astype(_i32) + 2 * BPAD, jnp.full((pad,), BPAD, _i32),
    ])
    rebase = jnp.where(jnp.arange(EPAD, dtype=_i32) < CORE0E, 0, BPAD)
    src = (src - rebase).reshape(EROWS, SUB)
    dst = (dst - rebase).reshape(EROWS, SUB)
    vals = jnp.concatenate([L0_val, L1_val, L2_val, jnp.zeros((pad,), _f32)])

    Xs = jnp.pad(jnp.stack([X0, X1, X2]), ((0, 0), (0, BPAD - NN), (0, 0)))
    W1s = jnp.stack([W01, W11, W21])
    W2s = jnp.stack([W02, W12, W22])
    W3s = jnp.stack([W03, W13, W23])
    b1s = jnp.stack([b01, b11, b21]).reshape(NBR, 1, HD)
    b2s = jnp.stack([b02, b12, b22])
    b3s = jnp.stack([b03, b13, b23]).reshape(NBR, 1, HD)
    bts = jnp.pad(jnp.stack([batch0, batch1, batch2]).astype(_i32),
                  ((0, 0), (0, BPAD - NN)),
                  constant_values=-1).reshape(NBR, 1, BPAD)

    y = _proj(Xs, W1s)
    p1 = _spmm(src, dst, vals, y.reshape(TPAD, HD))
    h1 = _mid_act(p1, p1, b1s, W2s)
    p2 = _spmm(src, dst, vals, h1.reshape(TPAD, HD))
    p3 = _spmm_rc(src, dst, vals, p2, b2s)
    return _pool(p3, b3s, bts, W3s, Wout, bout.reshape(1, OUTD))
